# baseline (speedup 1.0000x reference)
import functools

import jax
import jax.numpy as jnp
from jax import lax
from jax.experimental import pallas as pl
from jax.experimental.pallas import tpu as pltpu
from jax.experimental.pallas import tpu_sc as plsc

N = 10000
NP = 10240
D = 128
E = 320000
W = 128
NWIN = 2560
EP = NWIN * W
HWIN = NWIN // 16
AWIN = NWIN // 32
PAD0 = 10016


def _fill(ref, n16, value):
    v = jnp.full((16,), value, jnp.float32)

    def body(i, _):
        ref[pl.ds(i * 16, 16)] = v
        return 0

    lax.fori_loop(0, n16, body, 0)


def _rsqrt16(d):
    s = jnp.where(d < 16.0, 2.0,
                  jnp.where(d < 256.0, 8.0,
                            jnp.where(d < 4096.0, 32.0,
                                      jnp.where(d < 65536.0, 128.0, 512.0))))
    for _ in range(5):
        s = 0.5 * (s + d / s)
    return 1.0 / s


def _heron_slice(cnt_sh, row0, out_ref, tmp_ref):
    pltpu.sync_copy(cnt_sh.at[pl.ds(row0, 640)], tmp_ref)

    def body(k, _):
        d = jnp.maximum(tmp_ref[pl.ds(k * 16, 16)], 1.0)
        out_ref[pl.ds(k * 16, 16)] = _rsqrt16(d)
        return 0

    lax.fori_loop(0, 40, body, 0)


def _scale_rows(buf, svec, sbase, nrows16):
    def body(t, _):
        s16 = svec[pl.ds(sbase + t * 16, 16)]
        for i in range(16):
            r = t * 16 + i
            sc = s16[i]
            for k in range(8):
                buf[r, pl.ds(k * 16, 16)] = buf[r, pl.ds(k * 16, 16)] * sc
        return 0

    lax.fori_loop(0, nrows16, body, 0)


def _msgpass_body(src_hbm, dst_hbm, x_hbm, agg_hbm, cin_hbm, y0_hbm, y1_hbm,
                  cnt_out_sh, cnt_in_sh, agg_sh,
                  sidx, didx, rb0, rb1,
                  ones, zvec, cbuf, svec,
                  semh, semi, sem0, sem1, semw, sems0, sems1):
    c = lax.axis_index("c")
    s = lax.axis_index("s")
    wid = c * 16 + s
    zbase = s * 640

    _fill(ones, 8, 1.0)
    _fill(zvec, 40, 0.0)

    def zr_body(r, _):
        for k in range(8):
            rb0[r, pl.ds(k * 16, 16)] = jnp.zeros((16,), jnp.float32)
        return 0

    lax.fori_loop(0, 64, zr_body, 0)
    pltpu.sync_copy(zvec, cnt_out_sh.at[pl.ds(zbase, 640)])
    pltpu.sync_copy(zvec, cnt_in_sh.at[pl.ds(zbase, 640)])

    def za_body(k, _):
        pltpu.sync_copy(rb0.at[pl.ds(0, 64)], agg_sh.at[pl.ds(zbase + k * 64, 64)])
        return 0

    lax.fori_loop(0, 10, za_body, 0)
    pltpu.async_copy(x_hbm.at[pl.ds(zbase, 128)], rb0, sem0)
    pltpu.async_copy(x_hbm.at[pl.ds(zbase + 128, 128)], rb1, sem1)
    plsc.subcore_barrier()

    def hbatch(b, _):
        pltpu.sync_copy(src_hbm.at[pl.ds(s * HWIN + b * 40, 40)], sidx)
        pltpu.sync_copy(dst_hbm.at[pl.ds(s * HWIN + b * 40, 40)], didx)
        my_in = (b // 2) == c

        def hist_body(w, _):
            pltpu.async_copy(ones, cnt_out_sh.at[sidx.at[w]], semh, add=True)

            @pl.when(my_in)
            def _():
                pltpu.async_copy(ones, cnt_in_sh.at[didx.at[w]], semi, add=True)

            @pl.when(w > 1)
            def _():
                pltpu.make_async_copy(ones, cnt_out_sh.at[sidx.at[w]], semh).wait()

                @pl.when(my_in)
                def _():
                    pltpu.make_async_copy(ones, cnt_in_sh.at[didx.at[w]], semi).wait()

            return 0

        lax.fori_loop(0, 40, hist_body, 0)
        for _ in range(2):
            pltpu.make_async_copy(ones, cnt_out_sh.at[sidx.at[0]], semh).wait()

        @pl.when(my_in)
        def _():
            for _ in range(2):
                pltpu.make_async_copy(ones, cnt_in_sh.at[didx.at[0]], semi).wait()

        return 0

    lax.fori_loop(0, 4, hbatch, 0)
    plsc.subcore_barrier()

    _heron_slice(cnt_out_sh, zbase, svec, cbuf)

    def phase2(y_hbm):
        pltpu.make_async_copy(x_hbm.at[pl.ds(zbase, 128)], rb0, sem0).wait()
        _scale_rows(rb0, svec, 0, 8)
        pltpu.sync_copy(rb0, y_hbm.at[pl.ds(zbase, 128)])
        pltpu.make_async_copy(x_hbm.at[pl.ds(zbase + 128, 128)], rb1,
                              sem1).wait()
        _scale_rows(rb1, svec, 128, 8)
        pltpu.sync_copy(rb1, y_hbm.at[pl.ds(zbase + 128, 128)])

        def chunk(ci, rows16, buf):
            row0 = zbase + ci * 128
            nr = rows16 * 16
            pltpu.sync_copy(x_hbm.at[pl.ds(row0, nr)], buf.at[pl.ds(0, nr)])
            _scale_rows(buf, svec, ci * 128, rows16)
            pltpu.sync_copy(buf.at[pl.ds(0, nr)], y_hbm.at[pl.ds(row0, nr)])

        def c_body(ci, _):
            row0 = zbase + ci * 128

            @pl.when(row0 + 128 <= N)
            def _():
                chunk(ci, 8, rb0)

            @pl.when((row0 < N) & (row0 + 128 > N))
            def _():
                chunk(ci, 1, rb1)

            return 0

        lax.fori_loop(2, 5, c_body, 0)

    @pl.when(c == 0)
    def _():
        phase2(y0_hbm)

    @pl.when(c == 1)
    def _():
        phase2(y1_hbm)

    plsc.subcore_barrier()

    def phase3(y_hbm):
        def abatch(b, _):
            pltpu.sync_copy(src_hbm.at[pl.ds(wid * AWIN + b * 40, 40)], sidx)
            pltpu.sync_copy(dst_hbm.at[pl.ds(wid * AWIN + b * 40, 40)], didx)
            pltpu.async_copy(y_hbm.at[sidx.at[0]], rb0, sem0)
            pltpu.async_copy(y_hbm.at[sidx.at[1]], rb1, sem1)

            def pair_body(p, _):
                w0 = 2 * p
                w1 = w0 + 1
                pltpu.make_async_copy(y_hbm.at[sidx.at[w0]], rb0, sem0).wait()
                pltpu.sync_copy(rb0, agg_sh.at[didx.at[w0]], add=True)

                @pl.when(w0 + 2 < 40)
                def _():
                    pltpu.async_copy(y_hbm.at[sidx.at[w0 + 2]], rb0, sem0)

                pltpu.make_async_copy(y_hbm.at[sidx.at[w1]], rb1, sem1).wait()
                pltpu.sync_copy(rb1, agg_sh.at[didx.at[w1]], add=True)

                @pl.when(w1 + 2 < 40)
                def _():
                    pltpu.async_copy(y_hbm.at[sidx.at[w1 + 2]], rb1, sem1)

                return 0

            lax.fori_loop(0, 20, pair_body, 0)
            return 0

        lax.fori_loop(0, 2, abatch, 0)

    @pl.when(c == 0)
    def _():
        phase3(y0_hbm)

    @pl.when(c == 1)
    def _():
        phase3(y1_hbm)

    plsc.subcore_barrier()

    pltpu.sync_copy(agg_sh.at[pl.ds(zbase, 640)],
                    agg_hbm.at[c, pl.ds(zbase, 640)])
    pltpu.sync_copy(cnt_in_sh.at[pl.ds(zbase, 640)],
                    cin_hbm.at[c, pl.ds(zbase, 640)])


def _dense_body(aggp_ref, cinp_ref, w1_ref, b1_ref, gw_ref, gb_ref, cw_ref,
                cb_ref, c2w_ref, c2b_ref, out_ref, gate_ref, hg_ref):
    cin = cinp_ref[0] + cinp_ref[1]
    s_in = jax.lax.rsqrt(jnp.maximum(cin, 1.0))[:, :, None]
    agg3 = (aggp_ref[0] + aggp_ref[1]) * s_in
    agg = lax.slice(agg3.reshape(NP, D), (0, 0), (N, D))
    h = jnp.dot(agg, w1_ref[...], preferred_element_type=jnp.float32)
    h = jnp.maximum(h + b1_ref[...][None, :], 0.0)
    z = jnp.dot(h, gw_ref[...], preferred_element_type=jnp.float32)
    z = z + gb_ref[...][None, :]
    m = jnp.max(z)
    e = jnp.exp(z - m)
    gate = e / jnp.sum(e)
    gate_ref[...] = gate
    hg = jnp.sum(gate * h, axis=0, keepdims=True)
    hg_ref[...] = hg
    a2 = jnp.dot(hg, cw_ref[...], preferred_element_type=jnp.float32)
    a2 = a2 + cb_ref[...][None, :]
    a3 = jnp.dot(a2, c2w_ref[...], preferred_element_type=jnp.float32)
    a3 = a3 + c2b_ref[...][None, :]
    out_ref[...] = 1.0 / (1.0 + jnp.exp(-a3))


def kernel(x, edge_index, W1, b1, gate_W, gate_b, cls_W, cls_b, cls2_W, cls2_b):
    pad = PAD0 + (jnp.arange(EP - E, dtype=jnp.int32) % 128)
    src2d = jnp.concatenate([edge_index[0], pad]).reshape(NWIN, W)
    dst2d = jnp.concatenate([edge_index[1], pad]).reshape(NWIN, W)

    mesh = plsc.VectorSubcoreMesh(core_axis_name="c", subcore_axis_name="s")

    msgpass = pl.kernel(
        _msgpass_body,
        out_type=(
            jax.ShapeDtypeStruct((2, NP, D), jnp.float32),
            jax.ShapeDtypeStruct((2, NP), jnp.float32),
            jax.ShapeDtypeStruct((NP, D), jnp.float32),
            jax.ShapeDtypeStruct((NP, D), jnp.float32),
        ),
        mesh=mesh,
        scratch_types=[
            pltpu.VMEM_SHARED((NP,), jnp.float32),
            pltpu.VMEM_SHARED((NP,), jnp.float32),
            pltpu.VMEM_SHARED((NP, D), jnp.float32),
            pltpu.VMEM((40, W), jnp.int32),
            pltpu.VMEM((40, W), jnp.int32),
            pltpu.VMEM((W, D), jnp.float32),
            pltpu.VMEM((W, D), jnp.float32),
            pltpu.VMEM((W,), jnp.float32),
            pltpu.VMEM((640,), jnp.float32),
            pltpu.VMEM((640,), jnp.float32),
            pltpu.VMEM((640,), jnp.float32),
            pltpu.SemaphoreType.DMA,
            pltpu.SemaphoreType.DMA,
            pltpu.SemaphoreType.DMA,
            pltpu.SemaphoreType.DMA,
            pltpu.SemaphoreType.DMA,
            pltpu.SemaphoreType.DMA,
            pltpu.SemaphoreType.DMA,
        ],
        name="gcn_msgpass",
    )
    aggp, cinp, _, _ = msgpass(src2d, dst2d, x)

    out, gate, hg = pl.pallas_call(
        _dense_body,
        out_shape=(
            jax.ShapeDtypeStruct((1, 2), jnp.float32),
            jax.ShapeDtypeStruct((N, 1), jnp.float32),
            jax.ShapeDtypeStruct((1, D), jnp.float32),
        ),
        name="gcn_dense_tail",
    )(aggp.reshape(2, NP // 128, 128, D), cinp.reshape(2, NP // 128, 128),
      W1, b1, gate_W, gate_b, cls_W, cls_b, cls2_W, cls2_b)

    return (out, gate, hg)


# speedup vs baseline: 1.0061x; 1.0061x over previous
import functools

import jax
import jax.numpy as jnp
from jax import lax
from jax.experimental import pallas as pl
from jax.experimental.pallas import tpu as pltpu
from jax.experimental.pallas import tpu_sc as plsc

N = 10000
NP = 10240
D = 128
E = 320000
W = 128
NWIN = 2560
EP = NWIN * W
HWIN = NWIN // 16
AWIN = NWIN // 32
PAD0 = 10016


def _fill(ref, n16, value):
    v = jnp.full((16,), value, jnp.float32)

    def body(i, _):
        ref[pl.ds(i * 16, 16)] = v
        return 0

    lax.fori_loop(0, n16, body, 0)


def _rsqrt16(d):
    s = jnp.where(d < 16.0, 2.0,
                  jnp.where(d < 256.0, 8.0,
                            jnp.where(d < 4096.0, 32.0,
                                      jnp.where(d < 65536.0, 128.0, 512.0))))
    for _ in range(5):
        s = 0.5 * (s + d / s)
    return 1.0 / s


def _heron_slice(cnt_sh, row0, out_ref, tmp_ref):
    pltpu.sync_copy(cnt_sh.at[pl.ds(row0, 640)], tmp_ref)

    def body(k, _):
        d = jnp.maximum(tmp_ref[pl.ds(k * 16, 16)], 1.0)
        out_ref[pl.ds(k * 16, 16)] = _rsqrt16(d)
        return 0

    lax.fori_loop(0, 40, body, 0)


def _scale_rows(buf, svec, sbase, nrows16):
    def body(t, _):
        s16 = svec[pl.ds(sbase + t * 16, 16)]
        for i in range(16):
            r = t * 16 + i
            sc = s16[i]
            for k in range(8):
                buf[r, pl.ds(k * 16, 16)] = buf[r, pl.ds(k * 16, 16)] * sc
        return 0

    lax.fori_loop(0, nrows16, body, 0)


def _msgpass_body(src_hbm, dst_hbm, x_hbm, agg_hbm, cin_hbm, y0_hbm, y1_hbm,
                  cnt_out_sh, cnt_in_sh, agg_sh,
                  sidx, didx, rb0, rb1,
                  ones, zvec, cbuf, svec,
                  semh, semi, sem0, sem1, semw, sems0, sems1):
    c = lax.axis_index("c")
    s = lax.axis_index("s")
    wid = c * 16 + s
    zbase = s * 640

    _fill(ones, 8, 1.0)
    _fill(zvec, 40, 0.0)

    def zr_body(r, _):
        for k in range(8):
            rb0[r, pl.ds(k * 16, 16)] = jnp.zeros((16,), jnp.float32)
        return 0

    lax.fori_loop(0, 64, zr_body, 0)
    pltpu.sync_copy(zvec, cnt_out_sh.at[pl.ds(zbase, 640)])
    pltpu.sync_copy(zvec, cnt_in_sh.at[pl.ds(zbase, 640)])

    def za_body(k, _):
        pltpu.sync_copy(rb0.at[pl.ds(0, 64)], agg_sh.at[pl.ds(zbase + k * 64, 64)])
        return 0

    lax.fori_loop(0, 10, za_body, 0)
    plsc.subcore_barrier()

    def hbatch(b, _):
        pltpu.sync_copy(src_hbm.at[pl.ds(s * HWIN + b * 40, 40)], sidx)
        pltpu.sync_copy(dst_hbm.at[pl.ds(s * HWIN + b * 40, 40)], didx)
        my_in = (b // 2) == c

        def hist_body(w, _):
            pltpu.async_copy(ones, cnt_out_sh.at[sidx.at[w]], semh, add=True)

            @pl.when(my_in)
            def _():
                pltpu.async_copy(ones, cnt_in_sh.at[didx.at[w]], semi, add=True)

            @pl.when(w > 1)
            def _():
                pltpu.make_async_copy(ones, cnt_out_sh.at[sidx.at[w]], semh).wait()

                @pl.when(my_in)
                def _():
                    pltpu.make_async_copy(ones, cnt_in_sh.at[didx.at[w]], semi).wait()

            return 0

        lax.fori_loop(0, 40, hist_body, 0)
        for _ in range(2):
            pltpu.make_async_copy(ones, cnt_out_sh.at[sidx.at[0]], semh).wait()

        @pl.when(my_in)
        def _():
            for _ in range(2):
                pltpu.make_async_copy(ones, cnt_in_sh.at[didx.at[0]], semi).wait()

        return 0

    lax.fori_loop(0, 4, hbatch, 0)
    plsc.subcore_barrier()

    _heron_slice(cnt_out_sh, zbase, svec, cbuf)

    def phase2(y_hbm):
        def chunk(ci, rows16, buf):
            row0 = zbase + ci * 128
            nr = rows16 * 16
            pltpu.sync_copy(x_hbm.at[pl.ds(row0, nr)], buf.at[pl.ds(0, nr)])
            _scale_rows(buf, svec, ci * 128, rows16)
            pltpu.sync_copy(buf.at[pl.ds(0, nr)], y_hbm.at[pl.ds(row0, nr)])

        def c_body(ci, _):
            row0 = zbase + ci * 128

            @pl.when(row0 + 128 <= N)
            def _():
                chunk(ci, 8, rb0)

            @pl.when((row0 < N) & (row0 + 128 > N))
            def _():
                chunk(ci, 1, rb1)

            return 0

        lax.fori_loop(0, 5, c_body, 0)

    @pl.when(c == 0)
    def _():
        phase2(y0_hbm)

    @pl.when(c == 1)
    def _():
        phase2(y1_hbm)

    plsc.subcore_barrier()

    def phase3(y_hbm):
        def abatch(b, _):
            pltpu.sync_copy(src_hbm.at[pl.ds(wid * AWIN + b * 40, 40)], sidx)
            pltpu.sync_copy(dst_hbm.at[pl.ds(wid * AWIN + b * 40, 40)], didx)
            pltpu.async_copy(y_hbm.at[sidx.at[0]], rb0, sem0)
            pltpu.async_copy(y_hbm.at[sidx.at[1]], rb1, sem1)

            def pair_body(p, _):
                w0 = 2 * p
                w1 = w0 + 1
                pltpu.make_async_copy(y_hbm.at[sidx.at[w0]], rb0, sem0).wait()
                pltpu.sync_copy(rb0, agg_sh.at[didx.at[w0]], add=True)

                @pl.when(w0 + 2 < 40)
                def _():
                    pltpu.async_copy(y_hbm.at[sidx.at[w0 + 2]], rb0, sem0)

                pltpu.make_async_copy(y_hbm.at[sidx.at[w1]], rb1, sem1).wait()
                pltpu.sync_copy(rb1, agg_sh.at[didx.at[w1]], add=True)

                @pl.when(w1 + 2 < 40)
                def _():
                    pltpu.async_copy(y_hbm.at[sidx.at[w1 + 2]], rb1, sem1)

                return 0

            lax.fori_loop(0, 20, pair_body, 0)
            return 0

        lax.fori_loop(0, 2, abatch, 0)

    @pl.when(c == 0)
    def _():
        phase3(y0_hbm)

    @pl.when(c == 1)
    def _():
        phase3(y1_hbm)

    plsc.subcore_barrier()

    pltpu.sync_copy(agg_sh.at[pl.ds(zbase, 640)],
                    agg_hbm.at[c, pl.ds(zbase, 640)])
    pltpu.sync_copy(cnt_in_sh.at[pl.ds(zbase, 640)],
                    cin_hbm.at[c, pl.ds(zbase, 640)])


def _dense_body(aggp_ref, cinp_ref, w1_ref, b1_ref, gw_ref, gb_ref, cw_ref,
                cb_ref, c2w_ref, c2b_ref, out_ref, gate_ref, hg_ref):
    cin = cinp_ref[0] + cinp_ref[1]
    s_in = jax.lax.rsqrt(jnp.maximum(cin, 1.0))[:, :, None]
    agg3 = (aggp_ref[0] + aggp_ref[1]) * s_in
    agg = lax.slice(agg3.reshape(NP, D), (0, 0), (N, D))
    h = jnp.dot(agg, w1_ref[...], preferred_element_type=jnp.float32)
    h = jnp.maximum(h + b1_ref[...][None, :], 0.0)
    z = jnp.dot(h, gw_ref[...], preferred_element_type=jnp.float32)
    z = z + gb_ref[...][None, :]
    m = jnp.max(z)
    e = jnp.exp(z - m)
    gate = e / jnp.sum(e)
    gate_ref[...] = gate
    hg = jnp.sum(gate * h, axis=0, keepdims=True)
    hg_ref[...] = hg
    a2 = jnp.dot(hg, cw_ref[...], preferred_element_type=jnp.float32)
    a2 = a2 + cb_ref[...][None, :]
    a3 = jnp.dot(a2, c2w_ref[...], preferred_element_type=jnp.float32)
    a3 = a3 + c2b_ref[...][None, :]
    out_ref[...] = 1.0 / (1.0 + jnp.exp(-a3))


def kernel(x, edge_index, W1, b1, gate_W, gate_b, cls_W, cls_b, cls2_W, cls2_b):
    pad = PAD0 + (jnp.arange(EP - E, dtype=jnp.int32) % 128)
    src2d = jnp.concatenate([edge_index[0], pad]).reshape(NWIN, W)
    dst2d = jnp.concatenate([edge_index[1], pad]).reshape(NWIN, W)

    mesh = plsc.VectorSubcoreMesh(core_axis_name="c", subcore_axis_name="s")

    msgpass = pl.kernel(
        _msgpass_body,
        out_type=(
            jax.ShapeDtypeStruct((2, NP, D), jnp.float32),
            jax.ShapeDtypeStruct((2, NP), jnp.float32),
            jax.ShapeDtypeStruct((NP, D), jnp.float32),
            jax.ShapeDtypeStruct((NP, D), jnp.float32),
        ),
        mesh=mesh,
        scratch_types=[
            pltpu.VMEM_SHARED((NP,), jnp.float32),
            pltpu.VMEM_SHARED((NP,), jnp.float32),
            pltpu.VMEM_SHARED((NP, D), jnp.float32),
            pltpu.VMEM((40, W), jnp.int32),
            pltpu.VMEM((40, W), jnp.int32),
            pltpu.VMEM((W, D), jnp.float32),
            pltpu.VMEM((W, D), jnp.float32),
            pltpu.VMEM((W,), jnp.float32),
            pltpu.VMEM((640,), jnp.float32),
            pltpu.VMEM((640,), jnp.float32),
            pltpu.VMEM((640,), jnp.float32),
            pltpu.SemaphoreType.DMA,
            pltpu.SemaphoreType.DMA,
            pltpu.SemaphoreType.DMA,
            pltpu.SemaphoreType.DMA,
            pltpu.SemaphoreType.DMA,
            pltpu.SemaphoreType.DMA,
            pltpu.SemaphoreType.DMA,
        ],
        name="gcn_msgpass",
    )
    aggp, cinp, _, _ = msgpass(src2d, dst2d, x)

    out, gate, hg = pl.pallas_call(
        _dense_body,
        out_shape=(
            jax.ShapeDtypeStruct((1, 2), jnp.float32),
            jax.ShapeDtypeStruct((N, 1), jnp.float32),
            jax.ShapeDtypeStruct((1, D), jnp.float32),
        ),
        name="gcn_dense_tail",
    )(aggp.reshape(2, NP // 128, 128, D), cinp.reshape(2, NP // 128, 128),
      W1, b1, gate_W, gate_b, cls_W, cls_b, cls2_W, cls2_b)

    return (out, gate, hg)
